# baseline (speedup 1.0000x reference)
import jax
import jax.numpy as jnp
from jax import lax
from jax.experimental import pallas as pl
from jax.experimental.pallas import tpu as pltpu
from jax.experimental.pallas import tpu_sc as plsc

_D = 26
_B = 16384
_F = 128
_NW = 32
_BPW = _B // _NW
_CHUNK = 64
_NCHUNK = _BPW // _CHUNK
_L = 16


def _body(x_hbm, dom_hbm, out_hbm, dom_v, ridx_v, rows_v, gsems, ssem):
    wid = lax.axis_index("s") * 2 + lax.axis_index("c")
    base = wid * _BPW

    pltpu.sync_copy(dom_hbm.at[pl.ds(base, _BPW)], dom_v)

    lane = lax.iota(jnp.int32, _L)
    vpc = _CHUNK // _L

    @pl.loop(0, _BPW // _L)
    def _(v):
        d = dom_v[pl.ds(v * _L, _L)]
        ridx_v[v // vpc, pl.ds((v % vpc) * _L, _L)] = (
            d * _B + (base + v * _L) + lane)
    def _gather(c):
        return pltpu.make_async_copy(
            x_hbm.at[ridx_v.at[c]], rows_v.at[c], gsems.at[c])

    def _store(c):
        return pltpu.make_async_copy(
            rows_v.at[c], out_hbm.at[pl.ds(base + c * _CHUNK, _CHUNK)], ssem)

    @pl.loop(0, _NCHUNK)
    def _(c):
        _gather(c).start()

    @pl.loop(0, _NCHUNK)
    def _(c):
        _gather(c).wait()
        _store(c).start()

    @pl.loop(0, _NCHUNK)
    def _(c):
        _store(c).wait()


@jax.jit
def kernel(X, sample_domain):
    x2 = X.reshape(_D * _B, _F)
    mesh = plsc.VectorSubcoreMesh(core_axis_name="c", subcore_axis_name="s")
    k = pl.kernel(
        _body,
        out_type=jax.ShapeDtypeStruct((_B, _F), jnp.float32),
        mesh=mesh,
        scratch_types=[
            pltpu.VMEM((_BPW,), jnp.int32),
            pltpu.VMEM((_NCHUNK, _CHUNK), jnp.int32),
            pltpu.VMEM((_NCHUNK, _CHUNK, _F), jnp.float32),
            pltpu.SemaphoreType.DMA((_NCHUNK,)),
            pltpu.SemaphoreType.DMA,
        ],
    )
    return k(x2, sample_domain)


# speedup vs baseline: 1.0055x; 1.0055x over previous
import jax
import jax.numpy as jnp
from jax import lax
from jax.experimental import pallas as pl
from jax.experimental.pallas import tpu as pltpu
from jax.experimental.pallas import tpu_sc as plsc

_D = 26
_B = 16384
_F = 128
_NW = 32
_BPW = _B // _NW
_CHUNK = 128
_NCHUNK = _BPW // _CHUNK
_L = 16


def _body(x_hbm, dom_hbm, out_hbm, dom_v, ridx_v, rows_v, gsems, ssem):
    wid = lax.axis_index("s") * 2 + lax.axis_index("c")
    base = wid * _BPW

    pltpu.sync_copy(dom_hbm.at[pl.ds(base, _BPW)], dom_v)

    lane = lax.iota(jnp.int32, _L)
    vpc = _CHUNK // _L

    @pl.loop(0, _BPW // _L)
    def _(v):
        d = dom_v[pl.ds(v * _L, _L)]
        ridx_v[v // vpc, pl.ds((v % vpc) * _L, _L)] = (
            d * _B + (base + v * _L) + lane)
    def _gather(c):
        return pltpu.make_async_copy(
            x_hbm.at[ridx_v.at[c]], rows_v.at[c], gsems.at[c])

    def _store(c):
        return pltpu.make_async_copy(
            rows_v.at[c], out_hbm.at[pl.ds(base + c * _CHUNK, _CHUNK)], ssem)

    @pl.loop(0, _NCHUNK)
    def _(c):
        _gather(c).start()

    @pl.loop(0, _NCHUNK)
    def _(c):
        _gather(c).wait()
        _store(c).start()

    @pl.loop(0, _NCHUNK)
    def _(c):
        _store(c).wait()


@jax.jit
def kernel(X, sample_domain):
    x2 = X.reshape(_D * _B, _F)
    mesh = plsc.VectorSubcoreMesh(core_axis_name="c", subcore_axis_name="s")
    k = pl.kernel(
        _body,
        out_type=jax.ShapeDtypeStruct((_B, _F), jnp.float32),
        mesh=mesh,
        scratch_types=[
            pltpu.VMEM((_BPW,), jnp.int32),
            pltpu.VMEM((_NCHUNK, _CHUNK), jnp.int32),
            pltpu.VMEM((_NCHUNK, _CHUNK, _F), jnp.float32),
            pltpu.SemaphoreType.DMA((_NCHUNK,)),
            pltpu.SemaphoreType.DMA,
        ],
    )
    return k(x2, sample_domain)


# speedup vs baseline: 1.0075x; 1.0020x over previous
import jax
import jax.numpy as jnp
from jax import lax
from jax.experimental import pallas as pl
from jax.experimental.pallas import tpu as pltpu
from jax.experimental.pallas import tpu_sc as plsc

_D = 26
_B = 16384
_F = 128
_NW = 32
_BPW = _B // _NW
_CHUNK = 128
_NCHUNK = _BPW // _CHUNK
_L = 16


def _body(x_hbm, dom_hbm, out_hbm, dom_v, ridx_v, rows_v, gsems, ssem):
    wid = lax.axis_index("s") * 2 + lax.axis_index("c")
    base = wid * _BPW

    pltpu.sync_copy(dom_hbm.at[pl.ds(base, _BPW)], dom_v)

    lane = lax.iota(jnp.int32, _L)
    vpc = _CHUNK // _L

    @pl.loop(0, _BPW // _L)
    def _(v):
        d = dom_v[pl.ds(v * _L, _L)]
        ridx_v[v // vpc, pl.ds((v % vpc) * _L, _L)] = (
            d * _B + (base + v * _L) + lane)
    def _gather(c):
        return pltpu.make_async_copy(
            x_hbm.at[ridx_v.at[c]], rows_v.at[c], gsems.at[c])

    def _store(c):
        return pltpu.make_async_copy(
            rows_v.at[c], out_hbm.at[pl.ds(base + c * _CHUNK, _CHUNK)], ssem)

    for c in range(_NCHUNK):
        _gather(c).start()

    for c in range(_NCHUNK):
        _gather(c).wait()
        _store(c).start()

    for c in range(_NCHUNK):
        _store(c).wait()


@jax.jit
def kernel(X, sample_domain):
    x2 = X.reshape(_D * _B, _F)
    mesh = plsc.VectorSubcoreMesh(core_axis_name="c", subcore_axis_name="s")
    k = pl.kernel(
        _body,
        out_type=jax.ShapeDtypeStruct((_B, _F), jnp.float32),
        mesh=mesh,
        scratch_types=[
            pltpu.VMEM((_BPW,), jnp.int32),
            pltpu.VMEM((_NCHUNK, _CHUNK), jnp.int32),
            pltpu.VMEM((_NCHUNK, _CHUNK, _F), jnp.float32),
            pltpu.SemaphoreType.DMA((_NCHUNK,)),
            pltpu.SemaphoreType.DMA,
        ],
    )
    return k(x2, sample_domain)
